# baseline (speedup 1.0000x reference)
import functools

import jax
import jax.numpy as jnp
from jax import lax
from jax.experimental import pallas as pl
from jax.experimental.pallas import tpu as pltpu
from jax.experimental.pallas import tpu_sc as plsc

N = 10000
E = 160000
D = 256
DH = 128
DOUT = 40

NC = 2
NS = 16
DEG_W = 16

DEG_B = 40
ACC_ROWS = 10240
ZCH = ACC_ROWS // NS // DEG_B
DEG_EB = 128
DEG_KG = 8
DEG_EPT = 5120
DEG_NG = DEG_EPT // (DEG_KG * DEG_EB)

E_PER_TILE = E // NS
EB = 80
ESTEPS = E_PER_TILE // EB
N_PAD = ACC_ROWS
ROWS_PER_TILE = N_PAD // NS

def _copy_row(mat, k, out, n):
    for j in range(n // 16):
        out[pl.ds(j * 16, 16)] = mat[k, pl.ds(j * 16, 16)]


def _deg_body(dstd_hbm, degp_hbm, acc, idx8, dk, ones_v, fill_v, dsem):
    c = lax.axis_index("c")
    s = lax.axis_index("s")
    t = c * NS + s

    def fill(i, _):
        fill_v[i, :] = jnp.zeros((DEG_W,), jnp.float32)
        return 0
    lax.fori_loop(0, DEG_B, fill, 0)
    def zchunk(j, _):
        pltpu.sync_copy(
            fill_v, acc.at[pl.ds(s * (ACC_ROWS // NS) + j * DEG_B, DEG_B), :])
        return 0
    lax.fori_loop(0, ZCH, zchunk, 0)
    def ofill(i, _):
        ones_v[i, :] = jnp.ones((DEG_W,), jnp.float32)
        return 0
    lax.fori_loop(0, DEG_EB, ofill, 0)
    plsc.subcore_barrier()

    def dgroup(g, _):
        pltpu.async_copy(dstd_hbm.at[t, g], idx8, dsem).wait()
        for b in range(DEG_KG):
            _copy_row(idx8, b, dk, DEG_EB)
            pltpu.sync_copy(ones_v, acc.at[dk], add=True)
        return 0
    lax.fori_loop(0, DEG_NG, dgroup, 0)
    plsc.subcore_barrier()

    @pl.when(s == 0)
    def _():
        pltpu.sync_copy(acc.at[pl.ds(0, N_PAD), :], degp_hbm.at[c])


@functools.cache
def _deg_call():
    mesh = plsc.VectorSubcoreMesh(
        core_axis_name="c", subcore_axis_name="s",
        num_cores=NC, num_subcores=NS)
    return pl.kernel(
        _deg_body,
        out_type=jax.ShapeDtypeStruct((NC, N_PAD, DEG_W), jnp.float32),
        mesh=mesh,
        scratch_types=[
            pltpu.VMEM_SHARED((ACC_ROWS, DEG_W), jnp.float32),
            pltpu.VMEM((DEG_KG, DEG_EB), jnp.int32),
            pltpu.VMEM((DEG_EB,), jnp.int32),
            pltpu.VMEM((DEG_EB, DEG_W), jnp.float32),
            pltpu.VMEM((DEG_B, DEG_W), jnp.float32),
            pltpu.SemaphoreType.DMA,
        ],
    )


KG = 4
EPT_PAD = 10240
NG = EPT_PAD // (KG * EB)


def _scat_body(z_hbm, srcm_hbm, dstm_hbm, s_hbm,
               acc, src_g, dst_g, src_k, dst_k, rows_v, isem, jsem, gsem):
    c = lax.axis_index("c")
    s = lax.axis_index("s")
    r0 = s * ROWS_PER_TILE
    pltpu.sync_copy(z_hbm.at[c, pl.ds(r0, ROWS_PER_TILE), :],
                    acc.at[pl.ds(r0, ROWS_PER_TILE), :])
    plsc.subcore_barrier()

    zc = z_hbm.at[c]

    def group(g, _):
        di = pltpu.async_copy(srcm_hbm.at[s, g], src_g, isem)
        dj = pltpu.async_copy(dstm_hbm.at[s, g], dst_g, jsem)
        di.wait()
        dg = []
        for k in range(KG):
            _copy_row(src_g, k, src_k[k], EB)
            dg.append(pltpu.async_copy(zc.at[src_k[k]], rows_v[k], gsem[k]))
        dj.wait()
        for k in range(KG):
            _copy_row(dst_g, k, dst_k[k], EB)
            dg[k].wait()
            pltpu.sync_copy(rows_v[k], acc.at[dst_k[k]], add=True)
        return 0
    lax.fori_loop(0, NG, group, 0)
    plsc.subcore_barrier()

    pltpu.sync_copy(acc.at[pl.ds(r0, ROWS_PER_TILE), :],
                    s_hbm.at[c, pl.ds(r0, ROWS_PER_TILE), :])


@functools.cache
def _scat_call():
    mesh = plsc.VectorSubcoreMesh(
        core_axis_name="c", subcore_axis_name="s",
        num_cores=NC, num_subcores=NS)
    return pl.kernel(
        _scat_body,
        out_type=jax.ShapeDtypeStruct((NC, N_PAD, DH), jnp.float32),
        mesh=mesh,
        scratch_types=[
            pltpu.VMEM_SHARED((N_PAD, DH), jnp.float32),
            pltpu.VMEM((KG, EB), jnp.int32),
            pltpu.VMEM((KG, EB), jnp.int32),
            [pltpu.VMEM((EB,), jnp.int32)] * KG,
            [pltpu.VMEM((EB,), jnp.int32)] * KG,
            [pltpu.VMEM((EB, DH), jnp.float32)] * KG,
            pltpu.SemaphoreType.DMA,
            pltpu.SemaphoreType.DMA,
            [pltpu.SemaphoreType.DMA] * KG,
        ],
    )


RB = 640
GRID = N_PAD // RB


def _dinv_of(degp_ref):
    deg = 1.0 + degp_ref[0][:, 0:1] + degp_ref[1][:, 0:1]
    return lax.rsqrt(deg)


def _tc_first_body(degp_ref, x_ref, w_ref, z_ref):
    dinv = _dinv_of(degp_ref)
    z = dinv * jnp.dot(x_ref[...], w_ref[...], preferred_element_type=jnp.float32)
    z_ref[0] = z[:, :DH]
    z_ref[1] = z[:, DH:]


def _tc_mid_body(degp_ref, s_ref, b_ref, w_ref, h_ref, zn_ref):
    dinv = _dinv_of(degp_ref)
    agg = jnp.concatenate([s_ref[0], s_ref[1]], axis=1)
    h = jnp.maximum(dinv * agg + b_ref[...][None, :], 0.0)
    h_ref[...] = h
    zn = dinv * jnp.dot(h, w_ref[...], preferred_element_type=jnp.float32)
    zn_ref[0] = zn[:, :DH]
    zn_ref[1] = zn[:, DH:]


def _tc_last_body(degp_ref, s_ref, b_ref, wc_ref, bc_ref, h_ref, y_ref):
    dinv = _dinv_of(degp_ref)
    agg = jnp.concatenate([s_ref[0], s_ref[1]], axis=1)
    h = jnp.maximum(dinv * agg + b_ref[...][None, :], 0.0)
    h_ref[...] = h
    y_ref[...] = (jnp.dot(h, wc_ref[...], preferred_element_type=jnp.float32)
                  + bc_ref[...][None, :])


_degp_spec = pl.BlockSpec((NC, RB, DEG_W), lambda i: (0, i, 0))
_half_spec = pl.BlockSpec((NC, RB, DH), lambda i: (0, i, 0))
_full_spec = pl.BlockSpec((RB, D), lambda i: (i, 0))
_w_spec = pl.BlockSpec((D, D), lambda i: (0, 0))
_b_spec = pl.BlockSpec((D,), lambda i: (0,))

_tc_first = pl.pallas_call(
    _tc_first_body,
    grid=(GRID,),
    in_specs=[_degp_spec, _full_spec, _w_spec],
    out_specs=_half_spec,
    out_shape=jax.ShapeDtypeStruct((NC, N_PAD, DH), jnp.float32),
)

_tc_mid = pl.pallas_call(
    _tc_mid_body,
    grid=(GRID,),
    in_specs=[_degp_spec, _half_spec, _b_spec, _w_spec],
    out_specs=[_full_spec, _half_spec],
    out_shape=[
        jax.ShapeDtypeStruct((N, D), jnp.float32),
        jax.ShapeDtypeStruct((NC, N_PAD, DH), jnp.float32),
    ],
)

_tc_last = pl.pallas_call(
    _tc_last_body,
    grid=(GRID,),
    in_specs=[_degp_spec, _half_spec, _b_spec,
              pl.BlockSpec((D, DOUT), lambda i: (0, 0)),
              pl.BlockSpec((DOUT,), lambda i: (0,))],
    out_specs=[_full_spec, pl.BlockSpec((RB, DOUT), lambda i: (i, 0))],
    out_shape=[
        jax.ShapeDtypeStruct((N, D), jnp.float32),
        jax.ShapeDtypeStruct((N, DOUT), jnp.float32),
    ],
)


def kernel(x, edge_index, W1, b1, W2, b2, W3, b3, Wc, bc):
    src = edge_index[0]
    dst = edge_index[1]
    pad = EPT_PAD - E_PER_TILE
    srcm = jnp.pad(src.reshape(NS, E_PER_TILE), ((0, 0), (0, pad)),
                   constant_values=0).reshape(NS, NG, KG, EB)
    dstm = jnp.pad(dst.reshape(NS, E_PER_TILE), ((0, 0), (0, pad)),
                   constant_values=N).reshape(NS, NG, KG, EB)
    dpad = DEG_EPT - E // (NC * NS)
    dstd = jnp.pad(dst.reshape(NC * NS, E // (NC * NS)), ((0, 0), (0, dpad)),
                   constant_values=N).reshape(NC * NS, DEG_NG, DEG_KG, DEG_EB)
    degp = _deg_call()(dstd)
    z1 = _tc_first(degp, x, W1)
    s1 = _scat_call()(z1, srcm, dstm)
    h1, z2 = _tc_mid(degp, s1, b1, W2)
    s2 = _scat_call()(z2, srcm, dstm)
    h2, z3 = _tc_mid(degp, s2, b2, W3)
    s3 = _scat_call()(z3, srcm, dstm)
    h3, y = _tc_last(degp, s3, b3, Wc, bc)
    return (h1, h2, h3, y)


# speedup vs baseline: 1.0078x; 1.0078x over previous
import functools

import jax
import jax.numpy as jnp
from jax import lax
from jax.experimental import pallas as pl
from jax.experimental.pallas import tpu as pltpu
from jax.experimental.pallas import tpu_sc as plsc

N = 10000
E = 160000
D = 256
DH = 128
DOUT = 40

NC = 2
NS = 16
DEG_W = 16

DEG_B = 40
ACC_ROWS = 10240
ZCH = ACC_ROWS // NS // DEG_B
DEG_EB = 128
DEG_KG = 8
DEG_EPT = 5120
DEG_NG = DEG_EPT // (DEG_KG * DEG_EB)

E_PER_TILE = E // NS
EB = 80
ESTEPS = E_PER_TILE // EB
N_PAD = ACC_ROWS
ROWS_PER_TILE = N_PAD // NS

def _copy_row(mat, k, out, n):
    for j in range(n // 16):
        out[pl.ds(j * 16, 16)] = mat[k, pl.ds(j * 16, 16)]


def _deg_body(dstd_hbm, degp_hbm, acc, idx8, dk, ones_v, fill_v, dsem):
    c = lax.axis_index("c")
    s = lax.axis_index("s")
    t = c * NS + s

    def fill(i, _):
        fill_v[i, :] = jnp.zeros((DEG_W,), jnp.float32)
        return 0
    lax.fori_loop(0, DEG_B, fill, 0)
    def zchunk(j, _):
        pltpu.sync_copy(
            fill_v, acc.at[pl.ds(s * (ACC_ROWS // NS) + j * DEG_B, DEG_B), :])
        return 0
    lax.fori_loop(0, ZCH, zchunk, 0)
    def ofill(i, _):
        ones_v[i, :] = jnp.ones((DEG_W,), jnp.float32)
        return 0
    lax.fori_loop(0, DEG_EB, ofill, 0)
    plsc.subcore_barrier()

    def dgroup(g, _):
        pltpu.async_copy(dstd_hbm.at[t, g], idx8, dsem).wait()
        for b in range(DEG_KG):
            _copy_row(idx8, b, dk, DEG_EB)
            pltpu.sync_copy(ones_v, acc.at[dk], add=True)
        return 0
    lax.fori_loop(0, DEG_NG, dgroup, 0)
    plsc.subcore_barrier()

    @pl.when(s == 0)
    def _():
        pltpu.sync_copy(acc.at[pl.ds(0, N_PAD), :], degp_hbm.at[c])


@functools.cache
def _deg_call():
    mesh = plsc.VectorSubcoreMesh(
        core_axis_name="c", subcore_axis_name="s",
        num_cores=NC, num_subcores=NS)
    return pl.kernel(
        _deg_body,
        out_type=jax.ShapeDtypeStruct((NC, N_PAD, DEG_W), jnp.float32),
        mesh=mesh,
        scratch_types=[
            pltpu.VMEM_SHARED((ACC_ROWS, DEG_W), jnp.float32),
            pltpu.VMEM((DEG_KG, DEG_EB), jnp.int32),
            pltpu.VMEM((DEG_EB,), jnp.int32),
            pltpu.VMEM((DEG_EB, DEG_W), jnp.float32),
            pltpu.VMEM((DEG_B, DEG_W), jnp.float32),
            pltpu.SemaphoreType.DMA,
        ],
    )


KG = 4
EPT_PAD = 10240
NG = EPT_PAD // (KG * EB)


def _scat_body(z_hbm, srcm_hbm, dstm_hbm, s_hbm,
               acc, src_g, dst_g, src_k, dst_k, rows_v, isem, jsem, gsem, ssem):
    c = lax.axis_index("c")
    s = lax.axis_index("s")
    r0 = s * ROWS_PER_TILE
    pltpu.sync_copy(z_hbm.at[c, pl.ds(r0, ROWS_PER_TILE), :],
                    acc.at[pl.ds(r0, ROWS_PER_TILE), :])
    plsc.subcore_barrier()

    zc = z_hbm.at[c]

    def group(g, _):
        di = pltpu.async_copy(srcm_hbm.at[s, g], src_g, isem)
        dj = pltpu.async_copy(dstm_hbm.at[s, g], dst_g, jsem)
        di.wait()
        dj.wait()
        for k in range(KG):
            _copy_row(src_g, k, src_k[k], EB)
            _copy_row(dst_g, k, dst_k[k], EB)
        dg = []
        for k in range(KG):
            dg.append(pltpu.async_copy(zc.at[src_k[k]], rows_v[k], gsem[k]))
        ds = []
        for k in range(KG):
            dg[k].wait()
            ds.append(pltpu.async_copy(rows_v[k], acc.at[dst_k[k]], ssem[k],
                                       add=True))
        for k in range(KG):
            ds[k].wait()
        return 0
    lax.fori_loop(0, NG, group, 0)
    plsc.subcore_barrier()

    pltpu.sync_copy(acc.at[pl.ds(r0, ROWS_PER_TILE), :],
                    s_hbm.at[c, pl.ds(r0, ROWS_PER_TILE), :])


@functools.cache
def _scat_call():
    mesh = plsc.VectorSubcoreMesh(
        core_axis_name="c", subcore_axis_name="s",
        num_cores=NC, num_subcores=NS)
    return pl.kernel(
        _scat_body,
        out_type=jax.ShapeDtypeStruct((NC, N_PAD, DH), jnp.float32),
        mesh=mesh,
        scratch_types=[
            pltpu.VMEM_SHARED((N_PAD, DH), jnp.float32),
            pltpu.VMEM((KG, EB), jnp.int32),
            pltpu.VMEM((KG, EB), jnp.int32),
            [pltpu.VMEM((EB,), jnp.int32)] * KG,
            [pltpu.VMEM((EB,), jnp.int32)] * KG,
            [pltpu.VMEM((EB, DH), jnp.float32)] * KG,
            pltpu.SemaphoreType.DMA,
            pltpu.SemaphoreType.DMA,
            [pltpu.SemaphoreType.DMA] * KG,
            [pltpu.SemaphoreType.DMA] * KG,
        ],
    )


RB = 640
GRID = N_PAD // RB


def _dinv_of(degp_ref):
    deg = 1.0 + degp_ref[0][:, 0:1] + degp_ref[1][:, 0:1]
    return lax.rsqrt(deg)


def _tc_first_body(degp_ref, x_ref, w_ref, z_ref):
    dinv = _dinv_of(degp_ref)
    z = dinv * jnp.dot(x_ref[...], w_ref[...], preferred_element_type=jnp.float32)
    z_ref[0] = z[:, :DH]
    z_ref[1] = z[:, DH:]


def _tc_mid_body(degp_ref, s_ref, b_ref, w_ref, h_ref, zn_ref):
    dinv = _dinv_of(degp_ref)
    agg = jnp.concatenate([s_ref[0], s_ref[1]], axis=1)
    h = jnp.maximum(dinv * agg + b_ref[...][None, :], 0.0)
    h_ref[...] = h
    zn = dinv * jnp.dot(h, w_ref[...], preferred_element_type=jnp.float32)
    zn_ref[0] = zn[:, :DH]
    zn_ref[1] = zn[:, DH:]


def _tc_last_body(degp_ref, s_ref, b_ref, wc_ref, bc_ref, h_ref, y_ref):
    dinv = _dinv_of(degp_ref)
    agg = jnp.concatenate([s_ref[0], s_ref[1]], axis=1)
    h = jnp.maximum(dinv * agg + b_ref[...][None, :], 0.0)
    h_ref[...] = h
    y_ref[...] = (jnp.dot(h, wc_ref[...], preferred_element_type=jnp.float32)
                  + bc_ref[...][None, :])


_degp_spec = pl.BlockSpec((NC, RB, DEG_W), lambda i: (0, i, 0))
_half_spec = pl.BlockSpec((NC, RB, DH), lambda i: (0, i, 0))
_full_spec = pl.BlockSpec((RB, D), lambda i: (i, 0))
_w_spec = pl.BlockSpec((D, D), lambda i: (0, 0))
_b_spec = pl.BlockSpec((D,), lambda i: (0,))

_tc_first = pl.pallas_call(
    _tc_first_body,
    grid=(GRID,),
    in_specs=[_degp_spec, _full_spec, _w_spec],
    out_specs=_half_spec,
    out_shape=jax.ShapeDtypeStruct((NC, N_PAD, DH), jnp.float32),
)

_tc_mid = pl.pallas_call(
    _tc_mid_body,
    grid=(GRID,),
    in_specs=[_degp_spec, _half_spec, _b_spec, _w_spec],
    out_specs=[_full_spec, _half_spec],
    out_shape=[
        jax.ShapeDtypeStruct((N, D), jnp.float32),
        jax.ShapeDtypeStruct((NC, N_PAD, DH), jnp.float32),
    ],
)

_tc_last = pl.pallas_call(
    _tc_last_body,
    grid=(GRID,),
    in_specs=[_degp_spec, _half_spec, _b_spec,
              pl.BlockSpec((D, DOUT), lambda i: (0, 0)),
              pl.BlockSpec((DOUT,), lambda i: (0,))],
    out_specs=[_full_spec, pl.BlockSpec((RB, DOUT), lambda i: (i, 0))],
    out_shape=[
        jax.ShapeDtypeStruct((N, D), jnp.float32),
        jax.ShapeDtypeStruct((N, DOUT), jnp.float32),
    ],
)


def kernel(x, edge_index, W1, b1, W2, b2, W3, b3, Wc, bc):
    src = edge_index[0]
    dst = edge_index[1]
    pad = EPT_PAD - E_PER_TILE
    srcm = jnp.pad(src.reshape(NS, E_PER_TILE), ((0, 0), (0, pad)),
                   constant_values=0).reshape(NS, NG, KG, EB)
    dstm = jnp.pad(dst.reshape(NS, E_PER_TILE), ((0, 0), (0, pad)),
                   constant_values=N).reshape(NS, NG, KG, EB)
    dpad = DEG_EPT - E // (NC * NS)
    dstd = jnp.pad(dst.reshape(NC * NS, E // (NC * NS)), ((0, 0), (0, dpad)),
                   constant_values=N).reshape(NC * NS, DEG_NG, DEG_KG, DEG_EB)
    degp = _deg_call()(dstd)
    z1 = _tc_first(degp, x, W1)
    s1 = _scat_call()(z1, srcm, dstm)
    h1, z2 = _tc_mid(degp, s1, b1, W2)
    s2 = _scat_call()(z2, srcm, dstm)
    h2, z3 = _tc_mid(degp, s2, b2, W3)
    s3 = _scat_call()(z3, srcm, dstm)
    h3, y = _tc_last(degp, s3, b3, Wc, bc)
    return (h1, h2, h3, y)
